# baseline (speedup 1.0000x reference)
import functools

import jax
import jax.numpy as jnp
from jax import lax
from jax.experimental import pallas as pl
from jax.experimental.pallas import tpu as pltpu
from jax.experimental.pallas import tpu_sc as plsc

_NUM_CORES = 2
_NUM_SUBCORES = 16
_INFLIGHT = 10
_GROUP = 4
_PACK = 16
_CP = 8
_VBLK = 16384
_SLAB = 65536
_RB = 2048


def _project(tT, w8, *, interpret=False):
    D, V = tT.shape
    nk = 128 // _CP
    jblocks = _SLAB // _RB
    vblocks = -(-V // _RB)

    def body(*refs):
        t_refs, w_ref, o_ref = refs[:nk], refs[nk], refs[nk + 1]
        pts = [
            jnp.dot(
                w_ref[...], t_refs[k][...], preferred_element_type=jnp.float32
            )
            for k in range(nk)
        ]
        o_ref[...] = jnp.concatenate(pts, axis=0).T

    def t_spec(k):
        return pl.BlockSpec(
            (D, _RB),
            lambda j, k=k: (0, jnp.minimum(k * jblocks + j, vblocks - 1)),
        )

    return pl.pallas_call(
        body,
        grid=(jblocks,),
        in_specs=[t_spec(k) for k in range(nk)]
        + [pl.BlockSpec((_CP, D), lambda j: (0, 0))],
        out_specs=pl.BlockSpec((_RB, nk * _CP), lambda j: (j, 0)),
        out_shape=jax.ShapeDtypeStruct((_SLAB, nk * _CP), jnp.float32),
        interpret=interpret,
    )(*([tT] * nk), w8.T)


def _pool(x2, p3, btile, S, B, *, interpret=False):
    nw = _NUM_CORES * _NUM_SUBCORES
    bpw = B // nw
    k = _INFLIGHT
    g = _GROUP
    ng = S // g
    assert x2.shape == (ng, g * B)
    assert B % nw == 0 and ng % k == 0 and btile.shape == (g * bpw, _CP)

    mesh = plsc.VectorSubcoreMesh(
        core_axis_name="c",
        subcore_axis_name="s",
        num_cores=_NUM_CORES,
        num_subcores=_NUM_SUBCORES,
    )

    @functools.partial(
        pl.kernel,
        out_type=jax.ShapeDtypeStruct((B, _CP), jnp.float32),
        mesh=mesh,
        interpret=interpret,
        compiler_params=pltpu.CompilerParams(use_tc_tiling_on_sc=False),
        scratch_types=[
            pltpu.VMEM((ng, g * bpw), jnp.int32),
            pltpu.VMEM((g * bpw, _CP), jnp.float32),
            pltpu.VMEM_SHARED((_NUM_SUBCORES * B // nw, _CP), jnp.float32),
            pltpu.VMEM((bpw,), jnp.int32),
            pltpu.SemaphoreType.DMA,
        ],
    )
    def pool(x_hbm, p3_hbm, btile_hbm, out_hbm, xv, acc, fold, idxv, sem):
        wid = lax.axis_index("s") * _NUM_CORES + lax.axis_index("c")
        base = wid * bpw
        for h in range(g):
            pltpu.sync_copy(
                x_hbm.at[:, pl.ds(h * B + base, bpw)],
                xv.at[:, pl.ds(h * bpw, bpw)],
            )
        pltpu.sync_copy(btile_hbm, acc)

        def trow(t, _):
            for j in range(g * bpw // 16):
                vv = xv[t, pl.ds(j * 16, 16)]
                xv[t, pl.ds(j * 16, 16)] = (vv & (_SLAB - 1)) * (
                    128 // _CP
                ) + (vv >> 16)
            return 0

        lax.fori_loop(0, ng, trow, 0)

        p8 = p3_hbm

        for j in range(k):
            pltpu.async_copy(p8.at[xv.at[j]], acc, sem, add=True)

        def chunk(c, _):
            g0 = c * k
            for j in range(k):
                pltpu.async_copy(p8.at[xv.at[g0 + k + j]], acc, sem, add=True)
            for _j in range(k):
                pltpu.make_async_copy(p8.at[xv.at[0]], acc, sem).wait()
            return 0

        lax.fori_loop(0, ng // k - 1, chunk, 0)
        for _j in range(k):
            pltpu.make_async_copy(p8.at[xv.at[0]], acc, sem).wait()
        if g == 1:
            pltpu.sync_copy(acc, out_hbm.at[pl.ds(base, bpw)])
        else:
            sid = lax.axis_index("s")
            sbase = sid * bpw

            def ib(j, _):
                idxv[pl.ds(j * 16, 16)] = (
                    lax.iota(jnp.int32, 16) + (sbase + j * 16)
                )
                return 0

            lax.fori_loop(0, bpw // 16, ib, 0)
            pltpu.sync_copy(acc.at[pl.ds(0, bpw)], fold.at[pl.ds(sbase, bpw)])
            for h in range(1, g):
                pltpu.sync_copy(
                    acc.at[pl.ds(h * bpw, bpw)], fold.at[idxv], add=True
                )
            pltpu.sync_copy(
                fold.at[pl.ds(sbase, bpw)], out_hbm.at[pl.ds(base, bpw)]
            )

    return pool(x2, p3, btile)


def kernel(x, table, W, b):
    S, B = x.shape
    D, C = W.shape
    bpw = B // (_NUM_CORES * _NUM_SUBCORES)
    V = table.shape[0]
    w8 = jnp.zeros((D, _CP), jnp.float32).at[:, :C].set(W * (1.0 / S))
    b8 = jnp.zeros((_CP,), jnp.float32).at[:C].set(b)
    btile = jnp.concatenate(
        [jnp.tile(b8, (bpw, 1)), jnp.zeros(((_GROUP - 1) * bpw, _CP))]
    )
    p3 = _project(table.T, w8)
    out8 = _pool(
        x.reshape(S // _GROUP, _GROUP * B),
        p3.reshape(_SLAB * (128 // _CP), _CP),
        btile,
        S,
        B,
    )
    return out8[:, :C]


# speedup vs baseline: 1.0127x; 1.0127x over previous
import functools

import jax
import jax.numpy as jnp
from jax import lax
from jax.experimental import pallas as pl
from jax.experimental.pallas import tpu as pltpu
from jax.experimental.pallas import tpu_sc as plsc

_NUM_CORES = 2
_NUM_SUBCORES = 16
_INFLIGHT = 10
_GROUP = 2
_PACK = 16
_CP = 8
_VBLK = 16384
_SLAB = 65536
_RB = 2048


def _project(tT, w8, *, interpret=False):
    D, V = tT.shape
    nk = 128 // _CP
    jblocks = _SLAB // _RB
    vblocks = -(-V // _RB)

    def body(*refs):
        t_refs, w_ref, o_ref = refs[:nk], refs[nk], refs[nk + 1]
        pts = [
            jnp.dot(
                w_ref[...], t_refs[k][...], preferred_element_type=jnp.float32
            )
            for k in range(nk)
        ]
        o_ref[...] = jnp.concatenate(pts, axis=0).T

    def t_spec(k):
        return pl.BlockSpec(
            (D, _RB),
            lambda j, k=k: (0, jnp.minimum(k * jblocks + j, vblocks - 1)),
        )

    return pl.pallas_call(
        body,
        grid=(jblocks,),
        in_specs=[t_spec(k) for k in range(nk)]
        + [pl.BlockSpec((_CP, D), lambda j: (0, 0))],
        out_specs=pl.BlockSpec((_RB, nk * _CP), lambda j: (j, 0)),
        out_shape=jax.ShapeDtypeStruct((_SLAB, nk * _CP), jnp.float32),
        interpret=interpret,
    )(*([tT] * nk), w8.T)


def _stage(x2, S, B, *, interpret=False):
    nw = _NUM_CORES * _NUM_SUBCORES
    bpw = B // nw
    g = _GROUP
    ng = S // g
    assert x2.shape == (ng, g * B)

    mesh = plsc.VectorSubcoreMesh(
        core_axis_name="c",
        subcore_axis_name="s",
        num_cores=_NUM_CORES,
        num_subcores=_NUM_SUBCORES,
    )

    @functools.partial(
        pl.kernel,
        out_type=jax.ShapeDtypeStruct((nw, ng, g * bpw), jnp.int32),
        mesh=mesh,
        interpret=interpret,
        compiler_params=pltpu.CompilerParams(use_tc_tiling_on_sc=False),
        scratch_types=[
            pltpu.VMEM((ng, g * bpw), jnp.int32),
        ],
    )
    def stage(x_hbm, xr_hbm, xv):
        wid = lax.axis_index("s") * _NUM_CORES + lax.axis_index("c")
        base = wid * bpw
        for h in range(g):
            pltpu.sync_copy(
                x_hbm.at[:, pl.ds(h * B + base, bpw)],
                xv.at[:, pl.ds(h * bpw, bpw)],
            )

        def trow(t, _):
            for j in range(g * bpw // 16):
                vv = xv[t, pl.ds(j * 16, 16)]
                xv[t, pl.ds(j * 16, 16)] = (vv & (_SLAB - 1)) * (
                    128 // _CP
                ) + (vv >> 16)
            return 0

        lax.fori_loop(0, ng, trow, 0)
        pltpu.sync_copy(xv, xr_hbm.at[wid])

    return stage(x2)


def _pool(xr, p3, btile, S, B, *, interpret=False):
    nw = _NUM_CORES * _NUM_SUBCORES
    bpw = B // nw
    k = _INFLIGHT
    g = _GROUP
    ng = S // g
    assert xr.shape == (nw, ng, g * bpw)
    assert B % nw == 0 and ng % k == 0 and btile.shape == (g * bpw, _CP)

    mesh = plsc.VectorSubcoreMesh(
        core_axis_name="c",
        subcore_axis_name="s",
        num_cores=_NUM_CORES,
        num_subcores=_NUM_SUBCORES,
    )

    @functools.partial(
        pl.kernel,
        out_type=jax.ShapeDtypeStruct((B, _CP), jnp.float32),
        mesh=mesh,
        interpret=interpret,
        compiler_params=pltpu.CompilerParams(use_tc_tiling_on_sc=False),
        scratch_types=[
            pltpu.VMEM((ng, g * bpw), jnp.int32),
            pltpu.VMEM((g * bpw, _CP), jnp.float32),
            pltpu.VMEM_SHARED((_NUM_SUBCORES * B // nw, _CP), jnp.float32),
            pltpu.VMEM((bpw,), jnp.int32),
            pltpu.SemaphoreType.DMA,
        ],
    )
    def pool(xr_hbm, p3_hbm, btile_hbm, out_hbm, xv, acc, fold, idxv, sem):
        wid = lax.axis_index("s") * _NUM_CORES + lax.axis_index("c")
        base = wid * bpw
        pltpu.sync_copy(xr_hbm.at[wid], xv)
        pltpu.sync_copy(btile_hbm, acc)

        p8 = p3_hbm

        for j in range(k):
            pltpu.async_copy(p8.at[xv.at[j]], acc, sem, add=True)

        def chunk(c, _):
            g0 = c * k
            for j in range(k):
                pltpu.async_copy(p8.at[xv.at[g0 + k + j]], acc, sem, add=True)
            for _j in range(k):
                pltpu.make_async_copy(p8.at[xv.at[0]], acc, sem).wait()
            return 0

        lax.fori_loop(0, ng // k - 1, chunk, 0)
        for _j in range(k):
            pltpu.make_async_copy(p8.at[xv.at[0]], acc, sem).wait()
        if g == 1:
            pltpu.sync_copy(acc, out_hbm.at[pl.ds(base, bpw)])
        else:
            sid = lax.axis_index("s")
            sbase = sid * bpw

            def ib(j, _):
                idxv[pl.ds(j * 16, 16)] = (
                    lax.iota(jnp.int32, 16) + (sbase + j * 16)
                )
                return 0

            lax.fori_loop(0, bpw // 16, ib, 0)
            pltpu.sync_copy(acc.at[pl.ds(0, bpw)], fold.at[pl.ds(sbase, bpw)])
            for h in range(1, g):
                pltpu.sync_copy(
                    acc.at[pl.ds(h * bpw, bpw)], fold.at[idxv], add=True
                )
            pltpu.sync_copy(
                fold.at[pl.ds(sbase, bpw)], out_hbm.at[pl.ds(base, bpw)]
            )

    return pool(xr, p3, btile)


def kernel(x, table, W, b):
    S, B = x.shape
    D, C = W.shape
    bpw = B // (_NUM_CORES * _NUM_SUBCORES)
    V = table.shape[0]
    w8 = jnp.zeros((D, _CP), jnp.float32).at[:, :C].set(W * (1.0 / S))
    b8 = jnp.zeros((_CP,), jnp.float32).at[:C].set(b)
    btile = jnp.concatenate(
        [jnp.tile(b8, (bpw, 1)), jnp.zeros(((_GROUP - 1) * bpw, _CP))]
    )
    xr = _stage(x.reshape(S // _GROUP, _GROUP * B), S, B)
    p3 = _project(table.T, w8)
    out8 = _pool(xr, p3.reshape(_SLAB * (128 // _CP), _CP), btile, S, B)
    return out8[:, :C]


# speedup vs baseline: 1.0242x; 1.0114x over previous
import functools

import jax
import jax.numpy as jnp
from jax import lax
from jax.experimental import pallas as pl
from jax.experimental.pallas import tpu as pltpu
from jax.experimental.pallas import tpu_sc as plsc

_NUM_CORES = 2
_NUM_SUBCORES = 16
_INFLIGHT = 20
_GROUP = 2
_PACK = 16
_CP = 8
_VBLK = 16384
_SLAB = 65536
_RB = 2048


def _project(tT, wT, inv_s, *, interpret=False):
    D, V = tT.shape
    C = wT.shape[0]
    nk = 128 // _CP
    jblocks = _SLAB // _RB
    vblocks = -(-V // _RB)

    def body(*refs):
        t_refs, w_ref, o_ref = refs[:nk], refs[nk], refs[nk + 1]
        w = w_ref[...] * inv_s
        zpad = jnp.zeros((_CP - C, _RB), jnp.float32)
        pts = []
        for k in range(nk):
            pt = jnp.dot(w, t_refs[k][...], preferred_element_type=jnp.float32)
            pts.append(pt)
            pts.append(zpad)
        o_ref[...] = jnp.concatenate(pts, axis=0).T

    def t_spec(k):
        return pl.BlockSpec(
            (D, _RB),
            lambda j, k=k: (0, jnp.minimum(k * jblocks + j, vblocks - 1)),
        )

    return pl.pallas_call(
        body,
        grid=(jblocks,),
        in_specs=[t_spec(k) for k in range(nk)]
        + [pl.BlockSpec((C, D), lambda j: (0, 0))],
        out_specs=pl.BlockSpec((_RB, nk * _CP), lambda j: (j, 0)),
        out_shape=jax.ShapeDtypeStruct((_SLAB, nk * _CP), jnp.float32),
        interpret=interpret,
    )(*([tT] * nk), wT)


def _stage(x2, S, B, *, interpret=False):
    nw = _NUM_CORES * _NUM_SUBCORES
    bpw = B // nw
    g = _GROUP
    ng = S // g
    assert x2.shape == (ng, g * B)

    mesh = plsc.VectorSubcoreMesh(
        core_axis_name="c",
        subcore_axis_name="s",
        num_cores=_NUM_CORES,
        num_subcores=_NUM_SUBCORES,
    )

    @functools.partial(
        pl.kernel,
        out_type=jax.ShapeDtypeStruct((nw, ng, g * bpw), jnp.int32),
        mesh=mesh,
        interpret=interpret,
        compiler_params=pltpu.CompilerParams(use_tc_tiling_on_sc=False),
        scratch_types=[
            pltpu.VMEM((ng, g * bpw), jnp.int32),
        ],
    )
    def stage(x_hbm, xr_hbm, xv):
        wid = lax.axis_index("s") * _NUM_CORES + lax.axis_index("c")
        base = wid * bpw
        for h in range(g):
            pltpu.sync_copy(
                x_hbm.at[:, pl.ds(h * B + base, bpw)],
                xv.at[:, pl.ds(h * bpw, bpw)],
            )

        def trow(t, _):
            for j in range(g * bpw // 16):
                vv = xv[t, pl.ds(j * 16, 16)]
                xv[t, pl.ds(j * 16, 16)] = (vv & (_SLAB - 1)) * (
                    128 // _CP
                ) + (vv >> 16)
            return 0

        lax.fori_loop(0, ng, trow, 0)
        pltpu.sync_copy(xv, xr_hbm.at[wid])

    return stage(x2)


def _pool(xr, p3, btile, S, B, *, interpret=False):
    nw = _NUM_CORES * _NUM_SUBCORES
    bpw = B // nw
    k = _INFLIGHT
    g = _GROUP
    ng = S // g
    assert xr.shape == (nw, ng, g * bpw)
    assert B % nw == 0 and ng % k == 0 and btile.shape == (g * bpw, _CP)

    mesh = plsc.VectorSubcoreMesh(
        core_axis_name="c",
        subcore_axis_name="s",
        num_cores=_NUM_CORES,
        num_subcores=_NUM_SUBCORES,
    )

    @functools.partial(
        pl.kernel,
        out_type=jax.ShapeDtypeStruct((B, _CP), jnp.float32),
        mesh=mesh,
        interpret=interpret,
        compiler_params=pltpu.CompilerParams(use_tc_tiling_on_sc=False),
        scratch_types=[
            pltpu.VMEM((ng, g * bpw), jnp.int32),
            pltpu.VMEM((g * bpw, _CP), jnp.float32),
            pltpu.VMEM_SHARED((_NUM_SUBCORES * B // nw, _CP), jnp.float32),
            pltpu.VMEM((bpw,), jnp.int32),
            pltpu.SemaphoreType.DMA,
        ],
    )
    def pool(xr_hbm, p3_hbm, btile_hbm, out_hbm, xv, acc, fold, idxv, sem):
        wid = lax.axis_index("s") * _NUM_CORES + lax.axis_index("c")
        base = wid * bpw
        pltpu.sync_copy(xr_hbm.at[wid], xv)
        pltpu.sync_copy(btile_hbm, acc)

        p8 = p3_hbm

        for j in range(k):
            pltpu.async_copy(p8.at[xv.at[j]], acc, sem, add=True)

        def chunk(c, _):
            g0 = c * k
            for j in range(k):
                pltpu.async_copy(p8.at[xv.at[g0 + k + j]], acc, sem, add=True)
            for _j in range(k):
                pltpu.make_async_copy(p8.at[xv.at[0]], acc, sem).wait()
            return 0

        lax.fori_loop(0, ng // k - 1, chunk, 0)
        for _j in range(k):
            pltpu.make_async_copy(p8.at[xv.at[0]], acc, sem).wait()
        if g == 1:
            pltpu.sync_copy(acc, out_hbm.at[pl.ds(base, bpw)])
        else:
            sid = lax.axis_index("s")
            sbase = sid * bpw

            def ib(j, _):
                idxv[pl.ds(j * 16, 16)] = (
                    lax.iota(jnp.int32, 16) + (sbase + j * 16)
                )
                return 0

            lax.fori_loop(0, bpw // 16, ib, 0)
            pltpu.sync_copy(acc.at[pl.ds(0, bpw)], fold.at[pl.ds(sbase, bpw)])
            for h in range(1, g):
                pltpu.sync_copy(
                    acc.at[pl.ds(h * bpw, bpw)], fold.at[idxv], add=True
                )
            pltpu.sync_copy(
                fold.at[pl.ds(sbase, bpw)], out_hbm.at[pl.ds(base, bpw)]
            )

    return pool(xr, p3, btile)


def kernel(x, table, W, b):
    S, B = x.shape
    D, C = W.shape
    bpw = B // (_NUM_CORES * _NUM_SUBCORES)
    b8 = jnp.zeros((_CP,), jnp.float32).at[:C].set(b)
    btile = jnp.concatenate(
        [jnp.tile(b8, (bpw, 1)), jnp.zeros(((_GROUP - 1) * bpw, _CP))]
    )
    xr = _stage(x.reshape(S // _GROUP, _GROUP * B), S, B)
    p3 = _project(table.T, W.T, 1.0 / S)
    out8 = _pool(xr, p3.reshape(_SLAB * (128 // _CP), _CP), btile, S, B)
    return out8[:, :C]


# speedup vs baseline: 1.0278x; 1.0035x over previous
import functools

import jax
import jax.numpy as jnp
from jax import lax
from jax.experimental import pallas as pl
from jax.experimental.pallas import tpu as pltpu
from jax.experimental.pallas import tpu_sc as plsc

_NUM_CORES = 2
_NUM_SUBCORES = 16
_INFLIGHT = 20
_GROUP = 2
_PACK = 16
_CP = 8
_VBLK = 16384
_SLAB = 65536
_RB = 2048


def _project(tT, wT, inv_s, *, interpret=False):
    D, V = tT.shape
    C = wT.shape[0]
    nk = 128 // _CP
    jblocks = _SLAB // _RB
    vblocks = -(-V // _RB)

    def body(*refs):
        t_refs, w_ref, o_ref = refs[:nk], refs[nk], refs[nk + 1]
        w = w_ref[...] * inv_s
        zpad = jnp.zeros((_CP - C, _RB), jnp.float32)
        pts = []
        for k in range(nk):
            pt = jnp.dot(w, t_refs[k][...], preferred_element_type=jnp.float32)
            pts.append(pt)
            pts.append(zpad)
        o_ref[...] = jnp.concatenate(pts, axis=0).T

    def t_spec(k):
        return pl.BlockSpec(
            (D, _RB),
            lambda j, k=k: (0, jnp.minimum(k * jblocks + j, vblocks - 1)),
        )

    return pl.pallas_call(
        body,
        grid=(jblocks,),
        in_specs=[t_spec(k) for k in range(nk)]
        + [pl.BlockSpec((C, D), lambda j: (0, 0))],
        out_specs=pl.BlockSpec((_RB, nk * _CP), lambda j: (j, 0)),
        out_shape=jax.ShapeDtypeStruct((_SLAB, nk * _CP), jnp.float32),
        interpret=interpret,
    )(*([tT] * nk), wT)


def _stage(x2, S, B, *, interpret=False):
    nw = _NUM_CORES * _NUM_SUBCORES
    bpw = B // nw
    g = _GROUP
    ng = S // g
    assert x2.shape == (ng, g * B)

    mesh = plsc.VectorSubcoreMesh(
        core_axis_name="c",
        subcore_axis_name="s",
        num_cores=_NUM_CORES,
        num_subcores=_NUM_SUBCORES,
    )

    @functools.partial(
        pl.kernel,
        out_type=jax.ShapeDtypeStruct((nw, ng, g * bpw), jnp.int32),
        mesh=mesh,
        interpret=interpret,
        compiler_params=pltpu.CompilerParams(use_tc_tiling_on_sc=False),
        scratch_types=[
            pltpu.VMEM((ng, g * bpw), jnp.int32),
        ],
    )
    def stage(x_hbm, xr_hbm, xv):
        wid = lax.axis_index("s") * _NUM_CORES + lax.axis_index("c")
        base = wid * bpw
        for h in range(g):
            pltpu.sync_copy(
                x_hbm.at[:, pl.ds(h * B + base, bpw)],
                xv.at[:, pl.ds(h * bpw, bpw)],
            )

        def trow(t, _):
            for j in range(g * bpw // 16):
                vv = xv[t, pl.ds(j * 16, 16)]
                xv[t, pl.ds(j * 16, 16)] = (vv & (_SLAB - 1)) * (
                    128 // _CP
                ) + (vv >> 16)
            return 0

        lax.fori_loop(0, ng, trow, 0)
        pltpu.sync_copy(xv, xr_hbm.at[wid])

    return stage(x2)


def _pool(xr, p3, btile, S, B, C, *, interpret=False):
    nw = _NUM_CORES * _NUM_SUBCORES
    bpw = B // nw
    k = _INFLIGHT
    g = _GROUP
    ng = S // g
    assert xr.shape == (nw, ng, g * bpw)
    assert B % nw == 0 and ng % k == 0 and btile.shape == (g * bpw, _CP)

    mesh = plsc.VectorSubcoreMesh(
        core_axis_name="c",
        subcore_axis_name="s",
        num_cores=_NUM_CORES,
        num_subcores=_NUM_SUBCORES,
    )

    @functools.partial(
        pl.kernel,
        out_type=jax.ShapeDtypeStruct((B, _CP), jnp.float32),
        mesh=mesh,
        interpret=interpret,
        compiler_params=pltpu.CompilerParams(use_tc_tiling_on_sc=False),
        scratch_types=[
            pltpu.VMEM((ng, g * bpw), jnp.int32),
            pltpu.VMEM((g * bpw, _CP), jnp.float32),
            pltpu.VMEM_SHARED((_NUM_SUBCORES * B // nw, _CP), jnp.float32),
            pltpu.VMEM((bpw,), jnp.int32),
            pltpu.SemaphoreType.DMA,
        ],
    )
    def pool(xr_hbm, p3_hbm, btile_hbm, out_hbm, xv, acc, fold, idxv, sem):
        wid = lax.axis_index("s") * _NUM_CORES + lax.axis_index("c")
        base = wid * bpw
        pltpu.sync_copy(xr_hbm.at[wid], xv)
        pltpu.sync_copy(btile_hbm, acc)

        p8 = p3_hbm

        for j in range(k):
            pltpu.async_copy(p8.at[xv.at[j]], acc, sem, add=True)

        def chunk(c, _):
            g0 = c * k
            for j in range(k):
                pltpu.async_copy(p8.at[xv.at[g0 + k + j]], acc, sem, add=True)
            for _j in range(k):
                pltpu.make_async_copy(p8.at[xv.at[0]], acc, sem).wait()
            return 0

        lax.fori_loop(0, ng // k - 1, chunk, 0)
        for _j in range(k):
            pltpu.make_async_copy(p8.at[xv.at[0]], acc, sem).wait()
        sid = lax.axis_index("s")
        sbase = sid * bpw

        def ib(j, _):
            idxv[pl.ds(j * 16, 16)] = (
                lax.iota(jnp.int32, 16) + (sbase + j * 16)
            )
            return 0

        lax.fori_loop(0, bpw // 16, ib, 0)
        pltpu.sync_copy(acc.at[pl.ds(0, bpw)], fold.at[pl.ds(sbase, bpw)])
        for h in range(1, g):
            pltpu.sync_copy(
                acc.at[pl.ds(h * bpw, bpw)], fold.at[idxv], add=True
            )
        pltpu.sync_copy(
            fold.at[pl.ds(sbase, bpw)], out_hbm.at[pl.ds(base, bpw)]
        )

    return pool(xr, p3, btile)


def kernel(x, table, W, b):
    S, B = x.shape
    D, C = W.shape
    bpw = B // (_NUM_CORES * _NUM_SUBCORES)
    b8 = jnp.zeros((_CP,), jnp.float32).at[:C].set(b)
    btile = jnp.concatenate(
        [jnp.tile(b8, (bpw, 1)), jnp.zeros(((_GROUP - 1) * bpw, _CP))]
    )
    xr = _stage(x.reshape(S // _GROUP, _GROUP * B), S, B)
    p3 = _project(table.T, W.T, 1.0 / S)
    out8 = _pool(xr, p3.reshape(_SLAB * (128 // _CP), _CP), btile, S, B, C)
    return out8[:, :C]
